# baseline (speedup 1.0000x reference)
import functools
import math

import jax
import jax.numpy as jnp
from jax import lax
from jax.experimental import pallas as pl
from jax.experimental.pallas import tpu as pltpu
from jax.experimental.pallas import tpu_sc as plsc

D_MODEL = 64
SCALE = math.sqrt(D_MODEL)

NC = 2
NS = 16
NW = NC * NS
LANES = 16

G = 128
NB = 4


@functools.lru_cache(maxsize=None)
def _make_kernel(BT, S, V, D):
    assert BT == NW * G and D == 64
    assert S % NB == 0 and S >= 2 * NB
    DT = D // 8

    mesh = plsc.VectorSubcoreMesh(core_axis_name="c", subcore_axis_name="s")

    @functools.partial(
        pl.kernel,
        mesh=mesh,
        out_type=jax.ShapeDtypeStruct((S, DT, NW, 8 * G), jnp.float32),
        scratch_types=(
            [pltpu.VMEM((S, G), jnp.int32)]
            + [pltpu.VMEM((G, D), jnp.float32) for _ in range(NB)]
            + [pltpu.VMEM((D * G,), jnp.float32) for _ in range(NB)]
            + [pltpu.SemaphoreType.DMA((NB,)), pltpu.SemaphoreType.DMA((NB,))]
        ),
        compiler_params=pltpu.CompilerParams(
            use_tc_tiling_on_sc=False, needs_layout_passes=False
        ),
    )
    def k(xt_hbm, lut_hbm, out_hbm, idx_v, *bufs):
        gbuf = bufs[:NB]
        tbuf = bufs[NB:2 * NB]
        gsem, ssem = bufs[2 * NB], bufs[2 * NB + 1]
        wid = lax.axis_index("s") * NC + lax.axis_index("c")
        b0 = wid * G
        pltpu.sync_copy(xt_hbm.at[:, pl.ds(b0, G)], idx_v)

        iota = lax.iota(jnp.int32, 16)
        ibase = [(iota + d0) * G for d0 in range(0, D, 16)]

        def g_copy(s, b):
            return pltpu.make_async_copy(
                lut_hbm.at[idx_v.at[s]], gbuf[b], gsem.at[b]
            )

        def s_copies(s, b):
            return [
                pltpu.make_async_copy(
                    tbuf[b].at[pl.ds(dt * 8 * G, 8 * G)],
                    out_hbm.at[s, dt, wid],
                    ssem.at[b],
                )
                for dt in range(DT)
            ]

        def transpose_scale(b):
            def row_body(r, _):
                rsplat = jnp.full((16,), 0, jnp.int32) + r
                for u in range(D // 16):
                    v = gbuf[b][r, pl.ds(u * 16, 16)]
                    plsc.store_scatter(tbuf[b], [ibase[u] + rsplat], v * SCALE)
                return 0

            lax.fori_loop(0, G, row_body, 0, unroll=2)

        def process(s, b, scatter_wait, gather_fire):
            g_copy(s, b).wait()
            if scatter_wait:
                for cp in s_copies(0, b):
                    cp.wait()
            transpose_scale(b)
            for cp in s_copies(s, b):
                cp.start()
            if gather_fire:
                g_copy(s + NB, b).start()

        for b in range(NB):
            g_copy(b, b).start()
        for b in range(NB):
            process(b, b, scatter_wait=False, gather_fire=True)

        def outer(o, _):
            s0 = o * NB
            for b in range(NB):
                process(s0 + b, b, scatter_wait=True, gather_fire=True)
            return 0

        lax.fori_loop(1, S // NB - 1, outer, 0)

        for b in range(NB):
            process(S - NB + b, b, scatter_wait=True, gather_fire=False)

        for b in range(NB):
            for cp in s_copies(0, b):
                cp.wait()

    return k


def kernel(x, lut):
    BT, S = x.shape
    V, D = lut.shape
    xt = jnp.swapaxes(x, 0, 1).astype(jnp.int32)
    out4 = _make_kernel(BT, S, V, D)(xt, lut)
    out5 = out4.reshape(S, D // 8, NW, 8, G)
    return out5.transpose(2, 4, 0, 1, 3).reshape(BT, S, D)


# speedup vs baseline: 2.3109x; 2.3109x over previous
import functools
import math

import jax
import jax.numpy as jnp
from jax import lax
from jax.experimental import pallas as pl
from jax.experimental.pallas import tpu as pltpu
from jax.experimental.pallas import tpu_sc as plsc

D_MODEL = 64
SCALE = math.sqrt(D_MODEL)

NC = 2
NS = 16
NW = NC * NS
LANES = 16

G = 128
NB = 4


@functools.lru_cache(maxsize=None)
def _make_kernel(BT, S, V, D):
    assert BT == NW * G and D == 64
    assert S % NB == 0 and S >= 2 * NB
    DT = D // 8

    mesh = plsc.VectorSubcoreMesh(core_axis_name="c", subcore_axis_name="s")

    @functools.partial(
        pl.kernel,
        mesh=mesh,
        out_type=jax.ShapeDtypeStruct((S, DT, NW, 8, G), jnp.float32),
        scratch_types=(
            [pltpu.VMEM((S, G), jnp.int32)]
            + [pltpu.VMEM((G, D), jnp.float32) for _ in range(NB)]
            + [pltpu.VMEM((D, G + 1), jnp.float32) for _ in range(NB)]
            + [pltpu.SemaphoreType.DMA((NB,)), pltpu.SemaphoreType.DMA((NB,))]
        ),
        compiler_params=pltpu.CompilerParams(
            use_tc_tiling_on_sc=False, needs_layout_passes=False
        ),
    )
    def k(xt_hbm, lut_hbm, out_hbm, idx_v, *bufs):
        gbuf = bufs[:NB]
        tbuf = bufs[NB:2 * NB]
        gsem, ssem = bufs[2 * NB], bufs[2 * NB + 1]
        wid = lax.axis_index("s") * NC + lax.axis_index("c")
        b0 = wid * G
        pltpu.sync_copy(xt_hbm.at[:, pl.ds(b0, G)], idx_v)

        iota = lax.iota(jnp.int32, 16)
        diota = [iota + d0 for d0 in range(0, D, 16)]

        def g_copy(s, b):
            return pltpu.make_async_copy(
                lut_hbm.at[idx_v.at[s]], gbuf[b], gsem.at[b]
            )

        def s_copies(s, b):
            return [
                pltpu.make_async_copy(
                    tbuf[b].at[pl.ds(dt * 8, 8), pl.ds(0, G)],
                    out_hbm.at[s, dt, wid],
                    ssem.at[b],
                )
                for dt in range(DT)
            ]

        def transpose_scale(b):
            @functools.partial(plsc.parallel_loop, 0, G, unroll=4)
            def row_body(r):
                rsplat = jnp.full((16,), 0, jnp.int32) + r
                for u in range(D // 16):
                    v = gbuf[b][r, pl.ds(u * 16, 16)]
                    plsc.store_scatter(tbuf[b], [diota[u], rsplat], v * SCALE)

        def process(s, b, scatter_wait, gather_fire):
            g_copy(s, b).wait()
            if scatter_wait:
                for cp in s_copies(0, b):
                    cp.wait()
            transpose_scale(b)
            for cp in s_copies(s, b):
                cp.start()
            if gather_fire:
                g_copy(s + NB, b).start()

        for b in range(NB):
            g_copy(b, b).start()
        for b in range(NB):
            process(b, b, scatter_wait=False, gather_fire=True)

        def outer(o, _):
            s0 = o * NB
            for b in range(NB):
                process(s0 + b, b, scatter_wait=True, gather_fire=True)
            return 0

        lax.fori_loop(1, S // NB - 1, outer, 0)

        for b in range(NB):
            process(S - NB + b, b, scatter_wait=True, gather_fire=False)

        for b in range(NB):
            for cp in s_copies(0, b):
                cp.wait()

    return k


def kernel(x, lut):
    BT, S = x.shape
    V, D = lut.shape
    xt = jnp.swapaxes(x, 0, 1).astype(jnp.int32)
    out5 = _make_kernel(BT, S, V, D)(xt, lut)
    return out5.transpose(2, 4, 0, 1, 3).reshape(BT, S, D)
